# Initial kernel scaffold; baseline (speedup 1.0000x reference)
import jax
import jax.numpy as jnp
from jax.experimental import pallas as pl


def kernel(s, v, edge_rbf, edge_vec_unit, W1, b1, W2, b2, W3, b3, W4, b4, gamma, beta, edge_index):
    raise NotImplementedError("write your pallas kernel here")


# speedup vs baseline: 20.3934x; 20.3934x over previous
import functools

import jax
import jax.numpy as jnp
from jax import lax
from jax.experimental import pallas as pl
from jax.experimental.pallas import tpu as pltpu
from jax.experimental.pallas import tpu_sc as plsc

_NC = 2
_NS = 16
_GK = 400


def _sc_gather(src, tables):
    (E,) = src.shape
    H = tables[0].shape[1]
    T = len(tables)
    nw = _NC * _NS
    per_w = E // nw
    assert E % nw == 0 and per_w % _GK == 0
    nchunk = per_w // _GK
    mesh = plsc.VectorSubcoreMesh(core_axis_name="c", subcore_axis_name="s")

    @functools.partial(
        pl.kernel,
        out_type=tuple(jax.ShapeDtypeStruct((E, H), jnp.float32) for _ in range(T)),
        mesh=mesh,
        scratch_types=[pltpu.VMEM((_GK,), jnp.int32),
                       pltpu.VMEM((_GK, H), jnp.float32)],
    )
    def gather_kernel(*refs):
        src_hbm = refs[0]
        t_hbm = refs[1:1 + T]
        o_hbm = refs[1 + T:1 + 2 * T]
        idx_v = refs[1 + 2 * T]
        rows_v = refs[2 + 2 * T]
        wid = lax.axis_index("s") * _NC + lax.axis_index("c")
        base = wid * per_w

        @pl.loop(0, nchunk)
        def _(i):
            off = base + i * _GK
            pltpu.sync_copy(src_hbm.at[pl.ds(off, _GK)], idx_v)
            for t in range(T):
                pltpu.sync_copy(t_hbm[t].at[idx_v], rows_v)
                pltpu.sync_copy(rows_v, o_hbm[t].at[pl.ds(off, _GK)])

    return gather_kernel(src, *tables)


def _sc_scatter(dst, msgs, zeros):
    (E,) = dst.shape
    H = msgs[0].shape[1]
    T = len(msgs)
    n_pad = zeros.shape[0]
    per_t = E // _NS
    gk = 200
    nchunk = per_t // gk
    rows_per_tile = n_pad // _NS
    assert E % _NS == 0 and per_t % gk == 0 and n_pad % (_NS * 8) == 0
    mesh = plsc.VectorSubcoreMesh(core_axis_name="c", subcore_axis_name="s")

    @functools.partial(
        pl.kernel,
        out_type=tuple(jax.ShapeDtypeStruct((n_pad, H), jnp.float32)
                       for _ in range(T)),
        mesh=mesh,
        scratch_types=[pltpu.VMEM_SHARED((n_pad, H), jnp.float32),
                       pltpu.VMEM((gk,), jnp.int32),
                       pltpu.VMEM((gk, H), jnp.float32)],
    )
    def scatter_kernel(*refs):
        dst_hbm = refs[0]
        m_hbm = refs[1:1 + T]
        z_hbm = refs[1 + T]
        o_hbm = refs[2 + T:2 + 2 * T]
        acc, idx_v, rows_v = refs[2 + 2 * T:]
        cid = lax.axis_index("c")
        sid = lax.axis_index("s")
        sl = pl.ds(sid * rows_per_tile, rows_per_tile)
        base = sid * per_t

        def one_pass(m, o, core):
            @pl.when(cid == core)
            def _():
                pltpu.sync_copy(z_hbm.at[sl], acc.at[sl])

            plsc.subcore_barrier()

            @pl.when(cid == core)
            def _():
                @pl.loop(0, nchunk)
                def _(i):
                    off = base + i * gk
                    pltpu.sync_copy(dst_hbm.at[pl.ds(off, gk)], idx_v)
                    pltpu.sync_copy(m.at[pl.ds(off, gk)], rows_v)
                    pltpu.sync_copy(rows_v, acc.at[idx_v], add=True)

            plsc.subcore_barrier()

            @pl.when(cid == core)
            def _():
                pltpu.sync_copy(acc.at[sl], o.at[sl])

        for t in range(T):
            one_pass(m_hbm[t], o_hbm[t], t % _NC)

    return scatter_kernel(dst, *msgs, zeros)


def _pre_body(s, w, b, o):
    o[...] = jnp.dot(s[...], w[...], preferred_element_type=jnp.float32) + b[...]


def _edge_body(zg, rbf, e0, e1, e2, g0, g1, g2, w1r, w2a, w2b, w2c,
               b2a, b2b, b2c, o_s, o0, o1, o2):
    z = zg[...] + jnp.dot(rbf[...], w1r[...], preferred_element_type=jnp.float32)
    h = z * jax.nn.sigmoid(z)
    a_ss = jnp.dot(h, w2a[...], preferred_element_type=jnp.float32) + b2a[...]
    a_sv = jnp.dot(h, w2b[...], preferred_element_type=jnp.float32) + b2b[...]
    a_vv = jnp.dot(h, w2c[...], preferred_element_type=jnp.float32) + b2c[...]
    o_s[...] = a_ss
    o0[...] = a_sv * e0[...] + a_vv * g0[...]
    o1[...] = a_sv * e1[...] + a_vv * g1[...]
    o2[...] = a_sv * e2[...] + a_vv * g2[...]


def _node_body(s, agg_s, v0, v1, v2, a0, a1, a2, w3a, w3b, b3, w4a, w4b,
               b4a, b4b, gam, bet, ln_o, o0, o1, o2):
    vn0 = v0[...] + a0[...]
    vn1 = v1[...] + a1[...]
    vn2 = v2[...] + a2[...]
    vnorm = jnp.sqrt(vn0 * vn0 + vn1 * vn1 + vn2 * vn2)
    sa = s[...] + agg_s[...]
    z = (jnp.dot(sa, w3a[...], preferred_element_type=jnp.float32)
         + jnp.dot(vnorm, w3b[...], preferred_element_type=jnp.float32)
         + b3[...])
    h = z * jax.nn.sigmoid(z)
    delta = jnp.dot(h, w4a[...], preferred_element_type=jnp.float32) + b4a[...]
    gate = jnp.dot(h, w4b[...], preferred_element_type=jnp.float32) + b4b[...]
    x = s[...] + delta
    mu = jnp.mean(x, axis=-1, keepdims=True)
    var = jnp.mean((x - mu) * (x - mu), axis=-1, keepdims=True)
    ln_o[...] = (x - mu) * jax.lax.rsqrt(var + 1e-5) * gam[...] + bet[...]
    o0[...] = gate * vn0
    o1[...] = gate * vn1
    o2[...] = gate * vn2


def kernel(s, v, edge_rbf, edge_vec_unit, W1, b1, W2, b2, W3, b3, W4, b4,
           gamma, beta, edge_index):
    N, H = s.shape
    E = edge_index.shape[1]
    f32 = jnp.float32
    n_pad = -(-N // 128) * 128

    src = edge_index[0]
    dst = edge_index[1]
    v0 = v[:, 0, :]
    v1 = v[:, 1, :]
    v2 = v[:, 2, :]
    e0 = edge_vec_unit[:, 0:1]
    e1 = edge_vec_unit[:, 1:2]
    e2 = edge_vec_unit[:, 2:3]

    w1s = W1[:, :H].T
    w1r = W1[:, H:].T
    w2a = W2[:H, :].T
    w2b = W2[H:2 * H, :].T
    w2c = W2[2 * H:, :].T
    b2a = b2[None, :H]
    b2b = b2[None, H:2 * H]
    b2c = b2[None, 2 * H:]
    w3a = W3[:, :H].T
    w3b = W3[:, H:].T
    w4a = W4[:H, :].T
    w4b = W4[H:, :].T
    b4a = b4[None, :H]
    b4b = b4[None, H:]

    bn = 1000
    zs = pl.pallas_call(
        _pre_body,
        grid=(N // bn,),
        in_specs=[pl.BlockSpec((bn, H), lambda i: (i, 0)),
                  pl.BlockSpec((H, H), lambda i: (0, 0)),
                  pl.BlockSpec((1, H), lambda i: (0, 0))],
        out_specs=pl.BlockSpec((bn, H), lambda i: (i, 0)),
        out_shape=jax.ShapeDtypeStruct((N, H), f32),
    )(s, w1s, b1[None, :])

    zg, g0, g1, g2 = _sc_gather(src, (zs, v0, v1, v2))

    be = 2000
    blk = lambda r, c: pl.BlockSpec((r, c), lambda i: (i, 0))
    full = lambda r, c: pl.BlockSpec((r, c), lambda i: (0, 0))
    msg = pl.pallas_call(
        _edge_body,
        grid=(E // be,),
        in_specs=[blk(be, H), blk(be, edge_rbf.shape[1]),
                  blk(be, 1), blk(be, 1), blk(be, 1),
                  blk(be, H), blk(be, H), blk(be, H),
                  full(edge_rbf.shape[1], H),
                  full(H, H), full(H, H), full(H, H),
                  full(1, H), full(1, H), full(1, H)],
        out_specs=[blk(be, H)] * 4,
        out_shape=tuple(jax.ShapeDtypeStruct((E, H), f32) for _ in range(4)),
    )(zg, edge_rbf, e0, e1, e2, g0, g1, g2, w1r, w2a, w2b, w2c, b2a, b2b, b2c)

    zeros = jnp.zeros((n_pad, H), f32)
    agg_s, agg0, agg1, agg2 = _sc_scatter(dst, msg, zeros)

    ln, ov0, ov1, ov2 = pl.pallas_call(
        _node_body,
        grid=(N // bn,),
        in_specs=[blk(bn, H)] * 8 + [full(H, H), full(H, H), full(1, H),
                                     full(H, H), full(H, H), full(1, H),
                                     full(1, H), full(1, H), full(1, H)],
        out_specs=[blk(bn, H)] * 4,
        out_shape=tuple(jax.ShapeDtypeStruct((N, H), f32) for _ in range(4)),
    )(s, agg_s, v0, v1, v2, agg0, agg1, agg2, w3a, w3b, b3[None, :],
      w4a, w4b, b4a, b4b, gamma[None, :], beta[None, :])

    return (ln, jnp.stack([ov0, ov1, ov2], axis=1))


# speedup vs baseline: 23.5885x; 1.1567x over previous
import functools

import jax
import jax.numpy as jnp
from jax import lax
from jax.experimental import pallas as pl
from jax.experimental.pallas import tpu as pltpu
from jax.experimental.pallas import tpu_sc as plsc

_NC = 2
_NS = 16
_GK = 400


def _sc_gather(src, tables):
    (E,) = src.shape
    H = tables[0].shape[1]
    T = len(tables)
    nw = _NC * _NS
    per_w = E // nw
    nchunk = per_w // _GK
    assert E % nw == 0 and per_w % _GK == 0 and _GK % 8 == 0 and nchunk % 2 == 1
    mesh = plsc.VectorSubcoreMesh(core_axis_name="c", subcore_axis_name="s")

    @functools.partial(
        pl.kernel,
        out_type=tuple(jax.ShapeDtypeStruct((E, H), jnp.float32) for _ in range(T)),
        mesh=mesh,
        scratch_types=[pltpu.VMEM((per_w,), jnp.int32),
                       pltpu.VMEM((_GK, H), jnp.float32),
                       pltpu.VMEM((_GK, H), jnp.float32),
                       pltpu.SemaphoreType.DMA,
                       pltpu.SemaphoreType.DMA],
    )
    def gather_kernel(*refs):
        src_hbm = refs[0]
        t_hbm = refs[1:1 + T]
        o_hbm = refs[1 + T:1 + 2 * T]
        idx_all = refs[1 + 2 * T]
        rows = refs[2 + 2 * T:4 + 2 * T]
        sems = refs[4 + 2 * T:6 + 2 * T]
        wid = lax.axis_index("s") * _NC + lax.axis_index("c")
        base = wid * per_w
        pltpu.sync_copy(src_hbm.at[pl.ds(base, per_w)], idx_all)

        for t in range(T):
            tb = t_hbm[t]
            ob = o_hbm[t]

            def start(c, b, tb=tb):
                pltpu.async_copy(tb.at[idx_all.at[pl.ds(c * _GK, _GK)]],
                                 rows[b], sems[b])

            def finish(c, b, tb=tb, ob=ob):
                pltpu.make_async_copy(tb.at[pl.ds(0, _GK)], rows[b],
                                      sems[b]).wait()
                pltpu.sync_copy(rows[b], ob.at[pl.ds(base + c * _GK, _GK)])

            start(0, 0)

            @pl.loop(0, nchunk - 1, step=2)
            def _(i):
                start(i + 1, 1)
                finish(i, 0)
                start(i + 2, 0)
                finish(i + 1, 1)

            finish(nchunk - 1, 0)

    return gather_kernel(src, *tables)


def _sc_scatter(dst, msgs, zeros):
    (E,) = dst.shape
    H = msgs[0].shape[1]
    T = len(msgs)
    n_pad = zeros.shape[0]
    per_t = E // _NS
    gk = 160
    nchunk = per_t // gk
    wr = n_pad // 10
    assert (E % _NS == 0 and per_t % gk == 0 and gk % 8 == 0
            and nchunk % 2 == 1 and n_pad % 10 == 0 and wr % 8 == 0)
    mesh = plsc.VectorSubcoreMesh(core_axis_name="c", subcore_axis_name="s")

    @functools.partial(
        pl.kernel,
        out_type=tuple(jax.ShapeDtypeStruct((n_pad, H), jnp.float32)
                       for _ in range(T)),
        mesh=mesh,
        scratch_types=[pltpu.VMEM_SHARED((n_pad, H), jnp.float32),
                       pltpu.VMEM((gk,), jnp.int32),
                       pltpu.VMEM((gk,), jnp.int32),
                       pltpu.VMEM((gk, H), jnp.float32),
                       pltpu.VMEM((gk, H), jnp.float32),
                       pltpu.SemaphoreType.DMA,
                       pltpu.SemaphoreType.DMA,
                       pltpu.SemaphoreType.DMA,
                       pltpu.SemaphoreType.DMA],
    )
    def scatter_kernel(*refs):
        dst_hbm = refs[0]
        m_hbm = refs[1:1 + T]
        z_hbm = refs[1 + T]
        o_hbm = refs[2 + T:2 + 2 * T]
        acc = refs[2 + 2 * T]
        idxb = refs[3 + 2 * T:5 + 2 * T]
        rows = refs[5 + 2 * T:7 + 2 * T]
        isem = refs[7 + 2 * T:9 + 2 * T]
        rsem = refs[9 + 2 * T:11 + 2 * T]
        cid = lax.axis_index("c")
        sid = lax.axis_index("s")
        base = sid * per_t
        sl = pl.ds(sid * wr, wr)

        def one_pass(m, o, core):
            @pl.when((cid == core) & (sid < 10))
            def _():
                pltpu.sync_copy(z_hbm.at[sl], acc.at[sl])

            plsc.subcore_barrier()

            @pl.when(cid == core)
            def _():
                def start(c, b, m=m):
                    off = base + c * gk
                    pltpu.async_copy(dst_hbm.at[pl.ds(off, gk)], idxb[b],
                                     isem[b])
                    pltpu.async_copy(m.at[pl.ds(off, gk)], rows[b], rsem[b])

                def finish(c, b, m=m):
                    pltpu.make_async_copy(dst_hbm.at[pl.ds(0, gk)], idxb[b],
                                          isem[b]).wait()
                    pltpu.make_async_copy(m.at[pl.ds(0, gk)], rows[b],
                                          rsem[b]).wait()
                    pltpu.sync_copy(rows[b], acc.at[idxb[b]], add=True)

                start(0, 0)

                @pl.loop(0, nchunk - 1, step=2)
                def _(i):
                    start(i + 1, 1)
                    finish(i, 0)
                    start(i + 2, 0)
                    finish(i + 1, 1)

                finish(nchunk - 1, 0)

            plsc.subcore_barrier()

            @pl.when((cid == core) & (sid < 10))
            def _():
                pltpu.sync_copy(acc.at[sl], o.at[sl])

        for t in range(T):
            one_pass(m_hbm[t], o_hbm[t], t % _NC)

    return scatter_kernel(dst, *msgs, zeros)


def _pre_body(s, w, b, o):
    o[...] = jnp.dot(s[...], w[...], preferred_element_type=jnp.float32) + b[...]


def _edge_body(zg, rbf, e0, e1, e2, g0, g1, g2, w1r, w2a, w2b, w2c,
               b2a, b2b, b2c, o_s, o0, o1, o2):
    z = zg[...] + jnp.dot(rbf[...], w1r[...], preferred_element_type=jnp.float32)
    h = z * jax.nn.sigmoid(z)
    a_ss = jnp.dot(h, w2a[...], preferred_element_type=jnp.float32) + b2a[...]
    a_sv = jnp.dot(h, w2b[...], preferred_element_type=jnp.float32) + b2b[...]
    a_vv = jnp.dot(h, w2c[...], preferred_element_type=jnp.float32) + b2c[...]
    o_s[...] = a_ss
    o0[...] = a_sv * e0[...] + a_vv * g0[...]
    o1[...] = a_sv * e1[...] + a_vv * g1[...]
    o2[...] = a_sv * e2[...] + a_vv * g2[...]


def _node_body(s, agg_s, v0, v1, v2, a0, a1, a2, w3a, w3b, b3, w4a, w4b,
               b4a, b4b, gam, bet, ln_o, o0, o1, o2):
    vn0 = v0[...] + a0[...]
    vn1 = v1[...] + a1[...]
    vn2 = v2[...] + a2[...]
    vnorm = jnp.sqrt(vn0 * vn0 + vn1 * vn1 + vn2 * vn2)
    sa = s[...] + agg_s[...]
    z = (jnp.dot(sa, w3a[...], preferred_element_type=jnp.float32)
         + jnp.dot(vnorm, w3b[...], preferred_element_type=jnp.float32)
         + b3[...])
    h = z * jax.nn.sigmoid(z)
    delta = jnp.dot(h, w4a[...], preferred_element_type=jnp.float32) + b4a[...]
    gate = jnp.dot(h, w4b[...], preferred_element_type=jnp.float32) + b4b[...]
    x = s[...] + delta
    mu = jnp.mean(x, axis=-1, keepdims=True)
    var = jnp.mean((x - mu) * (x - mu), axis=-1, keepdims=True)
    ln_o[...] = (x - mu) * jax.lax.rsqrt(var + 1e-5) * gam[...] + bet[...]
    o0[...] = gate * vn0
    o1[...] = gate * vn1
    o2[...] = gate * vn2


def kernel(s, v, edge_rbf, edge_vec_unit, W1, b1, W2, b2, W3, b3, W4, b4,
           gamma, beta, edge_index):
    N, H = s.shape
    E = edge_index.shape[1]
    f32 = jnp.float32

    src = edge_index[0]
    dst = edge_index[1]
    v0 = v[:, 0, :]
    v1 = v[:, 1, :]
    v2 = v[:, 2, :]
    e0 = edge_vec_unit[:, 0:1]
    e1 = edge_vec_unit[:, 1:2]
    e2 = edge_vec_unit[:, 2:3]

    w1s = W1[:, :H].T
    w1r = W1[:, H:].T
    w2a = W2[:H, :].T
    w2b = W2[H:2 * H, :].T
    w2c = W2[2 * H:, :].T
    b2a = b2[None, :H]
    b2b = b2[None, H:2 * H]
    b2c = b2[None, 2 * H:]
    w3a = W3[:, :H].T
    w3b = W3[:, H:].T
    w4a = W4[:H, :].T
    w4b = W4[H:, :].T
    b4a = b4[None, :H]
    b4b = b4[None, H:]

    bn = 1000
    zs = pl.pallas_call(
        _pre_body,
        grid=(N // bn,),
        in_specs=[pl.BlockSpec((bn, H), lambda i: (i, 0)),
                  pl.BlockSpec((H, H), lambda i: (0, 0)),
                  pl.BlockSpec((1, H), lambda i: (0, 0))],
        out_specs=pl.BlockSpec((bn, H), lambda i: (i, 0)),
        out_shape=jax.ShapeDtypeStruct((N, H), f32),
    )(s, w1s, b1[None, :])

    zg, g0, g1, g2 = _sc_gather(src, (zs, v0, v1, v2))

    be = 2000
    blk = lambda r, c: pl.BlockSpec((r, c), lambda i: (i, 0))
    full = lambda r, c: pl.BlockSpec((r, c), lambda i: (0, 0))
    msg = pl.pallas_call(
        _edge_body,
        grid=(E // be,),
        in_specs=[blk(be, H), blk(be, edge_rbf.shape[1]),
                  blk(be, 1), blk(be, 1), blk(be, 1),
                  blk(be, H), blk(be, H), blk(be, H),
                  full(edge_rbf.shape[1], H),
                  full(H, H), full(H, H), full(H, H),
                  full(1, H), full(1, H), full(1, H)],
        out_specs=[blk(be, H)] * 4,
        out_shape=tuple(jax.ShapeDtypeStruct((E, H), f32) for _ in range(4)),
    )(zg, edge_rbf, e0, e1, e2, g0, g1, g2, w1r, w2a, w2b, w2c, b2a, b2b, b2c)

    zeros = jnp.zeros((N, H), f32)
    agg_s, agg0, agg1, agg2 = _sc_scatter(dst, msg, zeros)

    ln, ov0, ov1, ov2 = pl.pallas_call(
        _node_body,
        grid=(N // bn,),
        in_specs=[blk(bn, H)] * 8 + [full(H, H), full(H, H), full(1, H),
                                     full(H, H), full(H, H), full(1, H),
                                     full(1, H), full(1, H), full(1, H)],
        out_specs=[blk(bn, H)] * 4,
        out_shape=tuple(jax.ShapeDtypeStruct((N, H), f32) for _ in range(4)),
    )(s, agg_s, v0, v1, v2, agg0, agg1, agg2, w3a, w3b, b3[None, :],
      w4a, w4b, b4a, b4b, gamma[None, :], beta[None, :])

    return (ln, jnp.stack([ov0, ov1, ov2], axis=1))
